# Initial kernel scaffold; baseline (speedup 1.0000x reference)
import jax
import jax.numpy as jnp
from jax.experimental import pallas as pl


def kernel(x, table):
    raise NotImplementedError("write your pallas kernel here")


# speedup vs baseline: 2.9693x; 2.9693x over previous
import functools

import jax
import jax.numpy as jnp
from jax import lax
from jax.experimental import pallas as pl
from jax.experimental.pallas import tpu as pltpu
from jax.experimental.pallas import tpu_sc as plsc

NC = 2
NS = 16
NW = NC * NS
CHUNK = 128


@functools.partial(jax.jit, static_argnames=("n_chunks", "d"))
def _emb_lookup(xf, table, *, n_chunks, d):
    n_rows = NW * n_chunks * CHUNK

    mesh = plsc.VectorSubcoreMesh(
        core_axis_name="c", subcore_axis_name="s",
        num_cores=NC, num_subcores=NS,
    )

    @functools.partial(
        pl.kernel,
        out_type=jax.ShapeDtypeStruct((n_rows, d), jnp.float32),
        mesh=mesh,
        scratch_types=[
            pltpu.VMEM((n_chunks, CHUNK), jnp.int32),
            pltpu.VMEM((CHUNK, d), jnp.float32),
            pltpu.SemaphoreType.DMA,
        ],
    )
    def emb_kernel(table_hbm, idx_hbm, out_hbm, idx_v, rows_v, gsem):
        wid = lax.axis_index("s") * NC + lax.axis_index("c")
        pltpu.sync_copy(idx_hbm.at[wid], idx_v)

        def chunk_body(j, _):
            pltpu.async_copy(table_hbm.at[idx_v.at[j]], rows_v, gsem).wait()
            out_start = (wid * n_chunks + j) * CHUNK
            pltpu.sync_copy(rows_v, out_hbm.at[pl.ds(out_start, CHUNK)])
            return ()

        lax.fori_loop(0, n_chunks, chunk_body, ())

    return emb_kernel(table, xf)


def kernel(x, table):
    n = x.size
    d = table.shape[1]
    assert n % (NW * CHUNK) == 0
    n_chunks = n // (NW * CHUNK)
    xf = x.reshape(NW, n_chunks, CHUNK).astype(jnp.int32)
    out = _emb_lookup(xf, table, n_chunks=n_chunks, d=d)
    return out.reshape(*x.shape, d)


# speedup vs baseline: 3.3384x; 1.1243x over previous
import functools

import jax
import jax.numpy as jnp
from jax import lax
from jax.experimental import pallas as pl
from jax.experimental.pallas import tpu as pltpu
from jax.experimental.pallas import tpu_sc as plsc

NC = 2
NS = 16
NW = NC * NS
CHUNK = 128
M = 5
K = 2


@functools.partial(jax.jit, static_argnames=("n_chunks", "d"))
def _emb_lookup(xf, table, *, n_chunks, d):
    n_rows = NW * n_chunks * CHUNK
    assert n_chunks % M == 0 and n_chunks >= 2 * M

    mesh = plsc.VectorSubcoreMesh(
        core_axis_name="c", subcore_axis_name="s",
        num_cores=NC, num_subcores=NS,
    )

    @functools.partial(
        pl.kernel,
        out_type=jax.ShapeDtypeStruct((n_rows, d), jnp.float32),
        mesh=mesh,
        scratch_types=[
            pltpu.VMEM((n_chunks, CHUNK), jnp.int32),
            [pltpu.VMEM((CHUNK, d), jnp.float32) for _ in range(M)],
            [pltpu.SemaphoreType.DMA for _ in range(M)],
            [pltpu.SemaphoreType.DMA for _ in range(M)],
        ],
    )
    def emb_kernel(table_hbm, idx_hbm, out_hbm, idx_v, rows, gsem, wsem):
        wid = lax.axis_index("s") * NC + lax.axis_index("c")
        out_base = wid * n_chunks
        pltpu.sync_copy(idx_hbm.at[wid], idx_v)

        def gather(j, b):
            pltpu.async_copy(table_hbm.at[idx_v.at[j]], rows[b], gsem[b])

        def wait_gather(j, b):
            pltpu.make_async_copy(
                table_hbm.at[idx_v.at[j]], rows[b], gsem[b]).wait()

        def write(j, b):
            pltpu.async_copy(
                rows[b], out_hbm.at[pl.ds((out_base + j) * CHUNK, CHUNK)],
                wsem[b])

        def wait_write(j, b):
            pltpu.make_async_copy(
                rows[b], out_hbm.at[pl.ds((out_base + j) * CHUNK, CHUNK)],
                wsem[b]).wait()

        for b in range(M):
            gather(b, b)
            if b >= K:
                jj = b - K
                wait_gather(jj, jj)
                write(jj, jj)

        def round_body(r, _):
            for b in range(M):
                j = r * M + b
                wait_write(j - M, b)
                gather(j, b)
                bb = (b - K) % M
                wait_gather(j - K, bb)
                write(j - K, bb)
            return ()

        lax.fori_loop(1, n_chunks // M, round_body, ())

        for jj in range(n_chunks - K, n_chunks):
            bb = jj % M
            wait_gather(jj, bb)
            write(jj, bb)
        for b in range(M):
            wait_write(n_chunks - M + b, b)

    return emb_kernel(table, xf)


def kernel(x, table):
    n = x.size
    d = table.shape[1]
    assert n % (NW * CHUNK) == 0
    n_chunks = n // (NW * CHUNK)
    xf = x.reshape(NW, n_chunks, CHUNK).astype(jnp.int32)
    out = _emb_lookup(xf, table, n_chunks=n_chunks, d=d)
    return out.reshape(*x.shape, d)


# speedup vs baseline: 5.9565x; 1.7842x over previous
import functools

import jax
import jax.numpy as jnp
from jax import lax
from jax.experimental import pallas as pl
from jax.experimental.pallas import tpu as pltpu
from jax.experimental.pallas import tpu_sc as plsc

NC = 2
NS = 16
NW = NC * NS
M = 8
K = 4


@functools.partial(jax.jit, static_argnames=("nb", "s", "d"))
def _emb_lookup(xi, table, *, nb, s, d):
    mesh = plsc.VectorSubcoreMesh(
        core_axis_name="c", subcore_axis_name="s",
        num_cores=NC, num_subcores=NS,
    )

    @functools.partial(
        pl.kernel,
        out_type=jax.ShapeDtypeStruct((NW * nb, s, d), jnp.float32),
        mesh=mesh,
        scratch_types=[
            pltpu.VMEM((nb, s), jnp.int32),
            [pltpu.VMEM((s, d), jnp.float32) for _ in range(M)],
            [pltpu.SemaphoreType.DMA for _ in range(M)],
            [pltpu.SemaphoreType.DMA for _ in range(M)],
        ],
    )
    def emb_kernel(table_hbm, idx_hbm, out_hbm, idx_v, rows, gsem, wsem):
        wid = lax.axis_index("s") * NC + lax.axis_index("c")
        base = wid * nb
        pltpu.sync_copy(idx_hbm.at[pl.ds(base, nb)], idx_v)

        def gather(j, b):
            pltpu.async_copy(table_hbm.at[idx_v.at[j]], rows[b], gsem[b])

        def wait_gather(j, b):
            pltpu.make_async_copy(
                table_hbm.at[idx_v.at[j]], rows[b], gsem[b]).wait()

        def write(j, b):
            pltpu.async_copy(rows[b], out_hbm.at[base + j], wsem[b])

        def wait_write(j, b):
            pltpu.make_async_copy(
                rows[b], out_hbm.at[base + j], wsem[b]).wait()

        for b in range(M):
            gather(b, b)
            if b >= K:
                jj = b - K
                wait_gather(jj, jj)
                write(jj, jj)

        def round_body(r, _):
            for b in range(M):
                j = r * M + b
                wait_write(j - M, b)
                gather(j, b)
                bb = (b - K) % M
                wait_gather(j - K, bb)
                write(j - K, bb)
            return ()

        lax.fori_loop(1, nb // M, round_body, ())

        for jj in range(nb - K, nb):
            bb = jj % M
            wait_gather(jj, bb)
            write(jj, bb)
        for b in range(M):
            wait_write(nb - M + b, b)

    return emb_kernel(table, xi)


def kernel(x, table):
    n, s = x.shape
    d = table.shape[1]
    assert n % NW == 0
    nb = n // NW
    assert nb % M == 0 and nb >= 2 * M
    xi = x.astype(jnp.int32)
    return _emb_lookup(xi, table, nb=nb, s=s, d=d)


# speedup vs baseline: 5.9595x; 1.0005x over previous
import functools

import jax
import jax.numpy as jnp
from jax import lax
from jax.experimental import pallas as pl
from jax.experimental.pallas import tpu as pltpu
from jax.experimental.pallas import tpu_sc as plsc

NC = 2
NS = 16
NW = NC * NS
M = 8
K = 4


@functools.partial(jax.jit, static_argnames=("nb", "s", "d"))
def _emb_lookup(xi, table, *, nb, s, d):
    mesh = plsc.VectorSubcoreMesh(
        core_axis_name="c", subcore_axis_name="s",
        num_cores=NC, num_subcores=NS,
    )

    @functools.partial(
        pl.kernel,
        out_type=jax.ShapeDtypeStruct((NW * nb, s, d), jnp.float32),
        mesh=mesh,
        compiler_params=pltpu.CompilerParams(use_tc_tiling_on_sc=True),
        scratch_types=[
            pltpu.VMEM((nb, s), jnp.int32),
            [pltpu.VMEM((s, d), jnp.float32) for _ in range(M)],
            [pltpu.SemaphoreType.DMA for _ in range(M)],
            [pltpu.SemaphoreType.DMA for _ in range(M)],
        ],
    )
    def emb_kernel(table_hbm, idx_hbm, out_hbm, idx_v, rows, gsem, wsem):
        wid = lax.axis_index("s") * NC + lax.axis_index("c")
        base = wid * nb
        pltpu.sync_copy(idx_hbm.at[pl.ds(base, nb)], idx_v)

        def gather(j, b):
            pltpu.async_copy(table_hbm.at[idx_v.at[j]], rows[b], gsem[b])

        def wait_gather(j, b):
            pltpu.make_async_copy(
                table_hbm.at[idx_v.at[j]], rows[b], gsem[b]).wait()

        def write(j, b):
            pltpu.async_copy(rows[b], out_hbm.at[base + j], wsem[b])

        def wait_write(j, b):
            pltpu.make_async_copy(
                rows[b], out_hbm.at[base + j], wsem[b]).wait()

        for b in range(M):
            gather(b, b)
            if b >= K:
                jj = b - K
                wait_gather(jj, jj)
                write(jj, jj)

        def round_body(r, _):
            for b in range(M):
                j = r * M + b
                wait_write(j - M, b)
                gather(j, b)
                bb = (b - K) % M
                wait_gather(j - K, bb)
                write(j - K, bb)
            return ()

        lax.fori_loop(1, nb // M, round_body, ())

        for jj in range(nb - K, nb):
            bb = jj % M
            wait_gather(jj, bb)
            write(jj, bb)
        for b in range(M):
            wait_write(nb - M + b, b)

    return emb_kernel(table, xi)


def kernel(x, table):
    n, s = x.shape
    d = table.shape[1]
    assert n % NW == 0
    nb = n // NW
    assert nb % M == 0 and nb >= 2 * M
    xi = x.astype(jnp.int32)
    return _emb_lookup(xi, table, nb=nb, s=s, d=d)
